# baseline (speedup 1.0000x reference)
import jax
import jax.numpy as jnp
import numpy as np
from jax import lax
from jax.experimental import pallas as pl

RC = 5.0
NB = 16
CC = 64
NQ = 9
CMT = 576
BI = 16
OFFS = (0, 64, 256, 576)
MS = (1, 3, 5)
WID = RC / NB


def _silu(x):
    return x * jax.nn.sigmoid(x)


def _dsilu(x):
    s = jax.nn.sigmoid(x)
    return s * (1.0 + x * (1.0 - s))


def _pair_quantities(posn_ref, post, nmask, i0, bi, n):
    xi = posn_ref[0, pl.ds(i0, bi), 0:1]
    yi = posn_ref[0, pl.ds(i0, bi), 1:2]
    zi = posn_ref[0, pl.ds(i0, bi), 2:3]
    dx = post[0:1, :] - xi
    dy = post[1:2, :] - yi
    dz = post[2:3, :] - zi
    d2 = dx * dx + dy * dy + dz * dz + 1e-12
    dd = jnp.sqrt(d2)
    inv = 1.0 / dd
    row = lax.broadcasted_iota(jnp.int32, (bi, n), 0) + i0
    col = lax.broadcasted_iota(jnp.int32, (bi, n), 1)
    offdiag = jnp.where(row == col, 0.0, 1.0).astype(jnp.float32)
    incut = jnp.where(dd < RC, 1.0, 0.0).astype(jnp.float32)
    maskf = nmask * offdiag * incut
    env = 0.5 * (jnp.cos(jnp.pi * jnp.minimum(dd / RC, 1.0)) + 1.0) * maskf
    cent = (lax.broadcasted_iota(jnp.int32, (1, NB, 1), 1).astype(jnp.float32)
            * np.float32(RC / (NB - 1)))
    dev = dd[:, None, :] - cent
    rb = jnp.exp(-(dev * dev) / (2.0 * WID * WID))
    P = rb * env[:, None, :]
    xh = dx * inv
    yh = dy * inv
    zh = dz * inv
    Ys = jnp.concatenate(
        [
            jnp.ones((bi, 1, n), jnp.float32),
            xh[:, None, :], yh[:, None, :], zh[:, None, :],
            (xh * yh)[:, None, :], (yh * zh)[:, None, :],
            (3.0 * zh * zh - 1.0)[:, None, :],
            (xh * zh)[:, None, :], (xh * xh - yh * yh)[:, None, :],
        ],
        axis=1,
    )
    return dict(dd=dd, inv=inv, maskf=maskf, env=env, rb=rb, P=P,
                xh=xh, yh=yh, zh=zh, Ys=Ys, dev=dev)


def _fwd_body(posn_ref, post_ref, nmask_ref, f_ref, wexp_ref, vexp_ref,
              lw0_ref, lw1_ref, lw2_ref, lb_ref, ow_ref,
              fout_ref, u_ref, a_ref, e_ref):
    ib = pl.program_id(1)
    i0 = ib * BI
    post = post_ref[0]
    nm = nmask_ref[0]
    F = f_ref[0]
    n = F.shape[0]
    pq = _pair_quantities(posn_ref, post, nm, i0, BI, n)
    P, Ys = pq["P"], pq["Ys"]
    PY = P[:, :, None, :] * Ys[:, None, :, :]
    G = jnp.dot(P.reshape(BI * NB, n), F,
                preferred_element_type=jnp.float32)
    H = jnp.dot(PY.reshape(BI * NB * NQ, n), F[:, 0:CC],
                preferred_element_type=jnp.float32)
    t1 = jnp.sum(G.reshape(BI, NB, CMT) * wexp_ref[...][None], axis=1)
    H4 = H.reshape(BI, NB, NQ, CC)
    t2q = jnp.sum(H4 * vexp_ref[...][None], axis=1)
    t2 = jnp.concatenate([t2q[:, q, :] for q in range(NQ)], axis=1)
    conv = t1 + t2
    Fi = f_ref[0, pl.ds(i0, BI), :]
    lws = (lw0_ref, lw1_ref, lw2_ref)
    nf0 = None
    for l in range(3):
        sl = slice(OFFS[l], OFFS[l + 1])
        u = jnp.dot(conv[:, sl], lws[l][...],
                    preferred_element_type=jnp.float32) + lb_ref[0:1, sl]
        y = _silu(u)
        a = Fi[:, sl] + y
        nf = _silu(a)
        u_ref[0, :, sl] = u
        a_ref[0, :, sl] = a
        fout_ref[0, :, sl] = nf
        if l == 0:
            nf0 = nf
    eval_ = jnp.sum(nf0 * ow_ref[...])

    @pl.when(ib == 0)
    def _():
        e_ref[...] = jnp.full((1, 1, 1), 0.0, jnp.float32) + eval_

    @pl.when(ib > 0)
    def _():
        e_ref[...] = e_ref[...] + eval_


def _bwd_body(posn_ref, post_ref, nmask_ref, f_ref, ft_ref, u_ref, a_ref,
              gn_ref, wexp_ref, vexp_ref, lwt0_ref, lwt1_ref, lwt2_ref,
              gf_ref, gpt_ref):
    ib = pl.program_id(1)
    i0 = ib * BI
    post = post_ref[0]
    nm = nmask_ref[0]
    F = f_ref[0]
    FT = ft_ref[0]
    n = F.shape[0]
    pq = _pair_quantities(posn_ref, post, nm, i0, BI, n)
    P, Ys = pq["P"], pq["Ys"]

    a = a_ref[0]
    u = u_ref[0]
    gn = gn_ref[0]
    ga = gn * _dsilu(a)
    gu = ga * _dsilu(u)
    lwts = (lwt0_ref, lwt1_ref, lwt2_ref)
    gconv = jnp.concatenate(
        [jnp.dot(gu[:, OFFS[l]:OFFS[l + 1]], lwts[l][...],
                 preferred_element_type=jnp.float32) for l in range(3)],
        axis=1,
    )

    U1 = gconv[:, None, :] * wexp_ref[...][None]
    gP1 = jnp.dot(U1.reshape(BI * NB, CMT), FT,
                  preferred_element_type=jnp.float32)
    gf1 = lax.dot_general(P.reshape(BI * NB, n), U1.reshape(BI * NB, CMT),
                          (((0,), (0,)), ((), ())),
                          preferred_element_type=jnp.float32)
    gconv4 = jnp.concatenate(
        [gconv[:, q * CC:(q + 1) * CC][:, None, :] for q in range(NQ)], axis=1)
    U2 = gconv4[:, None, :, :] * vexp_ref[...][None]
    Z = jnp.dot(U2.reshape(BI * NB * NQ, CC), FT[0:CC, :],
                preferred_element_type=jnp.float32)
    Z4 = Z.reshape(BI, NB, NQ, n)
    gP = gP1.reshape(BI, NB, n) + jnp.sum(Z4 * Ys[:, None, :, :], axis=2)
    gY = jnp.sum(Z4 * P[:, :, None, :], axis=1)
    PY = P[:, :, None, :] * Ys[:, None, :, :]
    gs = lax.dot_general(PY.reshape(BI * NB * NQ, n), U2.reshape(BI * NB * NQ, CC),
                         (((0,), (0,)), ((), ())),
                         preferred_element_type=jnp.float32)

    dd, inv, env, rb, maskf = pq["dd"], pq["inv"], pq["env"], pq["rb"], pq["maskf"]
    xh, yh, zh, dev = pq["xh"], pq["yh"], pq["zh"], pq["dev"]
    rbp = rb * (-dev / (WID * WID))
    envp = (-0.5 * jnp.pi / RC) * jnp.sin(
        jnp.pi * jnp.minimum(dd / RC, 1.0)) * maskf
    gd = jnp.sum(gP * (rbp * env[:, None, :] + rb * envp[:, None, :]), axis=1)
    gxh = gY[:, 1] + gY[:, 4] * yh + gY[:, 7] * zh + 2.0 * gY[:, 8] * xh
    gyh = gY[:, 2] + gY[:, 4] * xh + gY[:, 5] * zh - 2.0 * gY[:, 8] * yh
    gzh = gY[:, 3] + gY[:, 5] * yh + 6.0 * gY[:, 6] * zh + gY[:, 7] * xh
    dotg = gxh * xh + gyh * yh + gzh * zh
    grx = gd * xh + (gxh - dotg * xh) * inv
    gry = gd * yh + (gyh - dotg * yh) * inv
    grz = gd * zh + (gzh - dotg * zh) * inv

    ones_bi = jnp.ones((1, BI), jnp.float32)
    ones_n = jnp.ones((1, n), jnp.float32)
    cn = (((1,), (1,)), ((), ()))
    colx = jnp.dot(ones_bi, grx, preferred_element_type=jnp.float32)
    coly = jnp.dot(ones_bi, gry, preferred_element_type=jnp.float32)
    colz = jnp.dot(ones_bi, grz, preferred_element_type=jnp.float32)
    rowx = lax.dot_general(ones_n, grx, cn,
                           preferred_element_type=jnp.float32)
    rowy = lax.dot_general(ones_n, gry, cn, preferred_element_type=jnp.float32)
    rowz = lax.dot_general(ones_n, grz, cn, preferred_element_type=jnp.float32)
    ri = lax.broadcasted_iota(jnp.int32, (BI, n), 0) + i0
    ci = lax.broadcasted_iota(jnp.int32, (BI, n), 1)
    oh = jnp.where(ri == ci, 1.0, 0.0).astype(jnp.float32)
    sx = jnp.dot(rowx, oh, preferred_element_type=jnp.float32)
    sy = jnp.dot(rowy, oh, preferred_element_type=jnp.float32)
    sz = jnp.dot(rowz, oh, preferred_element_type=jnp.float32)

    @pl.when(ib == 0)
    def _():
        gf_ref[0, :, :] = jnp.zeros((n, CMT), jnp.float32)
        gpt_ref[0, :, :] = jnp.zeros((3, n), jnp.float32)

    gf_ref[0, :, :] += gf1
    gf_ref[0, :, 0:CC] += gs
    gf_ref[0, pl.ds(i0, BI), :] += ga
    gpt_ref[0, 0:1, :] += colx - sx
    gpt_ref[0, 1:2, :] += coly - sy
    gpt_ref[0, 2:3, :] += colz - sz


def _emb_body(z_ref, emb_ref, f0_ref):
    zc = z_ref[0]
    n = zc.shape[0]
    io = lax.broadcasted_iota(jnp.int32, (n, 128), 1)
    oh = jnp.where(io == zc, 1.0, 0.0).astype(jnp.float32)
    f0 = jnp.dot(oh, emb_ref[...], preferred_element_type=jnp.float32)
    f0_ref[0, :, :] = jnp.concatenate(
        [f0, jnp.zeros((n, CMT - CC), jnp.float32)], axis=1)


def _fwd_layer(posn, post, nmaskf, F, wexp, vexp, lw0, lw1, lw2, lb, ow):
    B, N = F.shape[0], F.shape[1]
    NI = N // BI
    full = lambda b, i: (b, 0, 0)
    w2 = lambda b, i: (0, 0)
    return pl.pallas_call(
        _fwd_body,
        grid=(B, NI),
        in_specs=[
            pl.BlockSpec((1, N, 3), full),
            pl.BlockSpec((1, 3, N), full),
            pl.BlockSpec((1, BI, N), lambda b, i: (b, i, 0)),
            pl.BlockSpec((1, N, CMT), full),
            pl.BlockSpec((NB, CMT), w2),
            pl.BlockSpec((NB, NQ, CC), lambda b, i: (0, 0, 0)),
            pl.BlockSpec((64, 64), w2),
            pl.BlockSpec((192, 192), w2),
            pl.BlockSpec((320, 320), w2),
            pl.BlockSpec((1, CMT), w2),
            pl.BlockSpec((1, CC), w2),
        ],
        out_specs=[
            pl.BlockSpec((1, BI, CMT), lambda b, i: (b, i, 0)),
            pl.BlockSpec((1, BI, CMT), lambda b, i: (b, i, 0)),
            pl.BlockSpec((1, BI, CMT), lambda b, i: (b, i, 0)),
            pl.BlockSpec((1, 1, 1), full),
        ],
        out_shape=[
            jax.ShapeDtypeStruct((B, N, CMT), jnp.float32),
            jax.ShapeDtypeStruct((B, N, CMT), jnp.float32),
            jax.ShapeDtypeStruct((B, N, CMT), jnp.float32),
            jax.ShapeDtypeStruct((B, 1, 1), jnp.float32),
        ],
    )(posn, post, nmaskf, F, wexp, vexp, lw0, lw1, lw2, lb, ow)


def _bwd_layer(posn, post, nmaskf, F, FT, U, A, GN, wexp, vexp, lwt0, lwt1, lwt2):
    B, N = F.shape[0], F.shape[1]
    NI = N // BI
    full = lambda b, i: (b, 0, 0)
    blk = lambda b, i: (b, i, 0)
    w2 = lambda b, i: (0, 0)
    return pl.pallas_call(
        _bwd_body,
        grid=(B, NI),
        in_specs=[
            pl.BlockSpec((1, N, 3), full),
            pl.BlockSpec((1, 3, N), full),
            pl.BlockSpec((1, BI, N), blk),
            pl.BlockSpec((1, N, CMT), full),
            pl.BlockSpec((1, CMT, N), full),
            pl.BlockSpec((1, BI, CMT), blk),
            pl.BlockSpec((1, BI, CMT), blk),
            pl.BlockSpec((1, BI, CMT), blk),
            pl.BlockSpec((NB, CMT), w2),
            pl.BlockSpec((NB, NQ, CC), lambda b, i: (0, 0, 0)),
            pl.BlockSpec((64, 64), w2),
            pl.BlockSpec((192, 192), w2),
            pl.BlockSpec((320, 320), w2),
        ],
        out_specs=[
            pl.BlockSpec((1, N, CMT), full),
            pl.BlockSpec((1, 3, N), full),
        ],
        out_shape=[
            jax.ShapeDtypeStruct((B, N, CMT), jnp.float32),
            jax.ShapeDtypeStruct((B, 3, N), jnp.float32),
        ],
    )(posn, post, nmaskf, F, FT, U, A, GN, wexp, vexp, lwt0, lwt1, lwt2)


def _embed(z, embp):
    B, N = z.shape
    return pl.pallas_call(
        _emb_body,
        grid=(B,),
        in_specs=[
            pl.BlockSpec((1, N, 1), lambda b: (b, 0, 0)),
            pl.BlockSpec((128, CC), lambda b: (0, 0)),
        ],
        out_specs=pl.BlockSpec((1, N, CMT), lambda b: (b, 0, 0)),
        out_shape=jax.ShapeDtypeStruct((B, N, CMT), jnp.float32),
    )(z.reshape(B, N, 1).astype(jnp.int32), embp)


def _prep_weights(params):
    wexp, vexp, lbs = [], [], []
    blocks = []
    for blk in params["blocks"]:
        we = jnp.concatenate([jnp.tile(blk["W"][l], (1, MS[l])) for l in range(3)], axis=1)
        ve = jnp.stack([blk["V"][0]] + [blk["V"][1]] * 3 + [blk["V"][2]] * 5, axis=1)
        lw, lwt, lb = [], [], []
        for l in range(3):
            M = MS[l]
            perm = (jnp.arange(CC)[None, :] * M + jnp.arange(M)[:, None]).reshape(-1)
            w = blk["lw"][l][perm][:, perm]
            lw.append(w)
            lwt.append(w.T)
            lb.append(blk["lb"][l][perm])
        blocks.append(dict(wexp=we, vexp=ve, lw=lw, lwt=lwt,
                           lb=jnp.concatenate(lb).reshape(1, CMT)))
    return blocks


def kernel(z, pos, neighbor_mask, params):
    B, N, _ = pos.shape
    posn = pos.astype(jnp.float32)
    post = jnp.transpose(posn, (0, 2, 1))
    nmaskf = neighbor_mask.astype(jnp.float32)
    blocks = _prep_weights(params)
    embp = jnp.concatenate(
        [params["emb"], jnp.zeros((128 - params["emb"].shape[0], CC), jnp.float32)], axis=0)
    ow = params["out_w"].reshape(1, CC)

    F = _embed(z, embp)
    saves = []
    e = None
    for t in range(3):
        bw = blocks[t]
        Fn, U, A, e = _fwd_layer(posn, post, nmaskf, F, bw["wexp"], bw["vexp"],
                                 bw["lw"][0], bw["lw"][1], bw["lw"][2], bw["lb"], ow)
        saves.append((F, U, A))
        F = Fn
    E = e[:, 0, 0] + N * params["out_b"][0]

    GN = jnp.concatenate(
        [jnp.broadcast_to(params["out_w"][:, 0][None, None, :], (B, N, CC)),
         jnp.zeros((B, N, CMT - CC), jnp.float32)], axis=2)
    gpt_sum = jnp.zeros((B, 3, N), jnp.float32)
    for t in (2, 1, 0):
        bw = blocks[t]
        Fin, U, A = saves[t]
        FT = jnp.transpose(Fin, (0, 2, 1))
        GN, gpt = _bwd_layer(posn, post, nmaskf, Fin, FT, U, A, GN,
                             bw["wexp"], bw["vexp"],
                             bw["lwt"][0], bw["lwt"][1], bw["lwt"][2])
        gpt_sum = gpt_sum + gpt
    Fforce = -jnp.transpose(gpt_sum, (0, 2, 1))
    return (E, Fforce)


# speedup vs baseline: 3.2859x; 3.2859x over previous
import jax
import jax.numpy as jnp
import numpy as np
from jax import lax
from jax.experimental import pallas as pl

RC = 5.0
NB = 16
CC = 64
NQ = 9
CMT = 576
BI = 16
OFFS = (0, 64, 256, 576)
MS = (1, 3, 5)
WID = RC / NB


def _silu(x):
    return x * jax.nn.sigmoid(x)


def _dsilu(x):
    s = jax.nn.sigmoid(x)
    return s * (1.0 + x * (1.0 - s))


def _pair_quantities(posn_ref, post, nmask, i0, bi, n):
    xi = posn_ref[0, pl.ds(i0, bi), 0:1]
    yi = posn_ref[0, pl.ds(i0, bi), 1:2]
    zi = posn_ref[0, pl.ds(i0, bi), 2:3]
    dx = post[0:1, :] - xi
    dy = post[1:2, :] - yi
    dz = post[2:3, :] - zi
    d2 = dx * dx + dy * dy + dz * dz + 1e-12
    dd = jnp.sqrt(d2)
    inv = 1.0 / dd
    row = lax.broadcasted_iota(jnp.int32, (bi, n), 0) + i0
    col = lax.broadcasted_iota(jnp.int32, (bi, n), 1)
    offdiag = jnp.where(row == col, 0.0, 1.0).astype(jnp.float32)
    incut = jnp.where(dd < RC, 1.0, 0.0).astype(jnp.float32)
    maskf = nmask * offdiag * incut
    env = 0.5 * (jnp.cos(jnp.pi * jnp.minimum(dd / RC, 1.0)) + 1.0) * maskf
    cent = (lax.broadcasted_iota(jnp.int32, (NB, 1, 1), 0).astype(jnp.float32)
            * np.float32(RC / (NB - 1)))
    dev = dd[None, :, :] - cent
    rb = jnp.exp(-(dev * dev) / (2.0 * WID * WID))
    P = rb * env[None, :, :]
    xh = dx * inv
    yh = dy * inv
    zh = dz * inv
    Ys = jnp.concatenate(
        [
            jnp.ones((1, bi, n), jnp.float32),
            xh[None], yh[None], zh[None],
            (xh * yh)[None], (yh * zh)[None],
            (3.0 * zh * zh - 1.0)[None],
            (xh * zh)[None], (xh * xh - yh * yh)[None],
        ],
        axis=0,
    )
    return dict(dd=dd, inv=inv, maskf=maskf, env=env, rb=rb, P=P,
                xh=xh, yh=yh, zh=zh, Ys=Ys, dev=dev)


def _fwd_body(posn_ref, post_ref, nmask_ref, f_ref, wexp_ref, vexp_ref,
              lw0_ref, lw1_ref, lw2_ref, lb_ref, ow_ref,
              fout_ref, u_ref, a_ref, e_ref):
    ib = pl.program_id(1)
    i0 = ib * BI
    post = post_ref[0]
    nm = nmask_ref[0]
    F = f_ref[0]
    n = F.shape[0]
    pq = _pair_quantities(posn_ref, post, nm, i0, BI, n)
    P, Ys = pq["P"], pq["Ys"]
    PY = P[:, None, :, :] * Ys[None, :, :, :]
    G = jnp.dot(P.reshape(NB * BI, n), F,
                preferred_element_type=jnp.float32)
    H = jnp.dot(PY.reshape(NB * NQ * BI, n), F[:, 0:CC],
                preferred_element_type=jnp.float32)
    t1 = jnp.sum(G.reshape(NB, BI, CMT) * wexp_ref[...][:, None, :], axis=0)
    H4 = H.reshape(NB, NQ, BI, CC)
    t2q = jnp.sum(H4 * vexp_ref[...][:, :, None, :], axis=0)
    t2 = jnp.concatenate([t2q[q] for q in range(NQ)], axis=1)
    conv = t1 + t2
    Fi = f_ref[0, pl.ds(i0, BI), :]
    lws = (lw0_ref, lw1_ref, lw2_ref)
    nf0 = None
    for l in range(3):
        sl = slice(OFFS[l], OFFS[l + 1])
        u = jnp.dot(conv[:, sl], lws[l][...],
                    preferred_element_type=jnp.float32) + lb_ref[0:1, sl]
        y = _silu(u)
        a = Fi[:, sl] + y
        nf = _silu(a)
        u_ref[0, :, sl] = u
        a_ref[0, :, sl] = a
        fout_ref[0, :, sl] = nf
        if l == 0:
            nf0 = nf
    eval_ = jnp.sum(nf0 * ow_ref[...])

    @pl.when(ib == 0)
    def _():
        e_ref[...] = jnp.full((1, 1, 1), 0.0, jnp.float32) + eval_

    @pl.when(ib > 0)
    def _():
        e_ref[...] = e_ref[...] + eval_


def _bwd_body(posn_ref, post_ref, nmask_ref, f_ref, ft_ref, u_ref, a_ref,
              gn_ref, wexp_ref, vexp_ref, lwt0_ref, lwt1_ref, lwt2_ref,
              gf_ref, gpt_ref):
    ib = pl.program_id(1)
    i0 = ib * BI
    post = post_ref[0]
    nm = nmask_ref[0]
    F = f_ref[0]
    FT = ft_ref[0]
    n = F.shape[0]
    pq = _pair_quantities(posn_ref, post, nm, i0, BI, n)
    P, Ys = pq["P"], pq["Ys"]

    a = a_ref[0]
    u = u_ref[0]
    gn = gn_ref[0]
    ga = gn * _dsilu(a)
    gu = ga * _dsilu(u)
    lwts = (lwt0_ref, lwt1_ref, lwt2_ref)
    gconv = jnp.concatenate(
        [jnp.dot(gu[:, OFFS[l]:OFFS[l + 1]], lwts[l][...],
                 preferred_element_type=jnp.float32) for l in range(3)],
        axis=1,
    )

    U1 = wexp_ref[...][:, None, :] * gconv[None]
    gP1 = jnp.dot(U1.reshape(NB * BI, CMT), FT,
                  preferred_element_type=jnp.float32)
    gf1 = lax.dot_general(P.reshape(NB * BI, n), U1.reshape(NB * BI, CMT),
                          (((0,), (0,)), ((), ())),
                          preferred_element_type=jnp.float32)
    gconvq = jnp.concatenate(
        [gconv[None, :, q * CC:(q + 1) * CC] for q in range(NQ)], axis=0)
    U2 = vexp_ref[...][:, :, None, :] * gconvq[None]
    Z = jnp.dot(U2.reshape(NB * NQ * BI, CC), FT[0:CC, :],
                preferred_element_type=jnp.float32)
    Z4 = Z.reshape(NB, NQ, BI, n)
    gP = gP1.reshape(NB, BI, n) + jnp.sum(Z4 * Ys[None], axis=1)
    gY = jnp.sum(Z4 * P[:, None, :, :], axis=0)
    PY = P[:, None, :, :] * Ys[None]
    gs = lax.dot_general(PY.reshape(NB * NQ * BI, n), U2.reshape(NB * NQ * BI, CC),
                         (((0,), (0,)), ((), ())),
                         preferred_element_type=jnp.float32)

    dd, inv, env, rb, maskf = pq["dd"], pq["inv"], pq["env"], pq["rb"], pq["maskf"]
    xh, yh, zh, dev = pq["xh"], pq["yh"], pq["zh"], pq["dev"]
    rbp = rb * (-dev / (WID * WID))
    envp = (-0.5 * jnp.pi / RC) * jnp.sin(
        jnp.pi * jnp.minimum(dd / RC, 1.0)) * maskf
    gd = jnp.sum(gP * (rbp * env[None] + rb * envp[None]), axis=0)
    gxh = gY[1] + gY[4] * yh + gY[7] * zh + 2.0 * gY[8] * xh
    gyh = gY[2] + gY[4] * xh + gY[5] * zh - 2.0 * gY[8] * yh
    gzh = gY[3] + gY[5] * yh + 6.0 * gY[6] * zh + gY[7] * xh
    dotg = gxh * xh + gyh * yh + gzh * zh
    grx = gd * xh + (gxh - dotg * xh) * inv
    gry = gd * yh + (gyh - dotg * yh) * inv
    grz = gd * zh + (gzh - dotg * zh) * inv

    ones_bi = jnp.ones((1, BI), jnp.float32)
    ones_n = jnp.ones((1, n), jnp.float32)
    cn = (((1,), (1,)), ((), ()))
    colx = jnp.dot(ones_bi, grx, preferred_element_type=jnp.float32)
    coly = jnp.dot(ones_bi, gry, preferred_element_type=jnp.float32)
    colz = jnp.dot(ones_bi, grz, preferred_element_type=jnp.float32)
    rowx = lax.dot_general(ones_n, grx, cn,
                           preferred_element_type=jnp.float32)
    rowy = lax.dot_general(ones_n, gry, cn, preferred_element_type=jnp.float32)
    rowz = lax.dot_general(ones_n, grz, cn, preferred_element_type=jnp.float32)
    ri = lax.broadcasted_iota(jnp.int32, (BI, n), 0) + i0
    ci = lax.broadcasted_iota(jnp.int32, (BI, n), 1)
    oh = jnp.where(ri == ci, 1.0, 0.0).astype(jnp.float32)
    sx = jnp.dot(rowx, oh, preferred_element_type=jnp.float32)
    sy = jnp.dot(rowy, oh, preferred_element_type=jnp.float32)
    sz = jnp.dot(rowz, oh, preferred_element_type=jnp.float32)

    @pl.when(ib == 0)
    def _():
        gf_ref[0, :, :] = jnp.zeros((n, CMT), jnp.float32)
        gpt_ref[0, :, :] = jnp.zeros((3, n), jnp.float32)

    gf_ref[0, :, :] += gf1
    gf_ref[0, :, 0:CC] += gs
    gf_ref[0, pl.ds(i0, BI), :] += ga
    gpt_ref[0, 0:1, :] += colx - sx
    gpt_ref[0, 1:2, :] += coly - sy
    gpt_ref[0, 2:3, :] += colz - sz


def _emb_body(z_ref, emb_ref, f0_ref):
    zc = z_ref[0]
    n = zc.shape[0]
    io = lax.broadcasted_iota(jnp.int32, (n, 128), 1)
    oh = jnp.where(io == zc, 1.0, 0.0).astype(jnp.float32)
    f0 = jnp.dot(oh, emb_ref[...], preferred_element_type=jnp.float32)
    f0_ref[0, :, :] = jnp.concatenate(
        [f0, jnp.zeros((n, CMT - CC), jnp.float32)], axis=1)


def _fwd_layer(posn, post, nmaskf, F, wexp, vexp, lw0, lw1, lw2, lb, ow):
    B, N = F.shape[0], F.shape[1]
    NI = N // BI
    full = lambda b, i: (b, 0, 0)
    w2 = lambda b, i: (0, 0)
    return pl.pallas_call(
        _fwd_body,
        grid=(B, NI),
        in_specs=[
            pl.BlockSpec((1, N, 3), full),
            pl.BlockSpec((1, 3, N), full),
            pl.BlockSpec((1, BI, N), lambda b, i: (b, i, 0)),
            pl.BlockSpec((1, N, CMT), full),
            pl.BlockSpec((NB, CMT), w2),
            pl.BlockSpec((NB, NQ, CC), lambda b, i: (0, 0, 0)),
            pl.BlockSpec((64, 64), w2),
            pl.BlockSpec((192, 192), w2),
            pl.BlockSpec((320, 320), w2),
            pl.BlockSpec((1, CMT), w2),
            pl.BlockSpec((1, CC), w2),
        ],
        out_specs=[
            pl.BlockSpec((1, BI, CMT), lambda b, i: (b, i, 0)),
            pl.BlockSpec((1, BI, CMT), lambda b, i: (b, i, 0)),
            pl.BlockSpec((1, BI, CMT), lambda b, i: (b, i, 0)),
            pl.BlockSpec((1, 1, 1), full),
        ],
        out_shape=[
            jax.ShapeDtypeStruct((B, N, CMT), jnp.float32),
            jax.ShapeDtypeStruct((B, N, CMT), jnp.float32),
            jax.ShapeDtypeStruct((B, N, CMT), jnp.float32),
            jax.ShapeDtypeStruct((B, 1, 1), jnp.float32),
        ],
    )(posn, post, nmaskf, F, wexp, vexp, lw0, lw1, lw2, lb, ow)


def _bwd_layer(posn, post, nmaskf, F, FT, U, A, GN, wexp, vexp, lwt0, lwt1, lwt2):
    B, N = F.shape[0], F.shape[1]
    NI = N // BI
    full = lambda b, i: (b, 0, 0)
    blk = lambda b, i: (b, i, 0)
    w2 = lambda b, i: (0, 0)
    return pl.pallas_call(
        _bwd_body,
        grid=(B, NI),
        in_specs=[
            pl.BlockSpec((1, N, 3), full),
            pl.BlockSpec((1, 3, N), full),
            pl.BlockSpec((1, BI, N), blk),
            pl.BlockSpec((1, N, CMT), full),
            pl.BlockSpec((1, CMT, N), full),
            pl.BlockSpec((1, BI, CMT), blk),
            pl.BlockSpec((1, BI, CMT), blk),
            pl.BlockSpec((1, BI, CMT), blk),
            pl.BlockSpec((NB, CMT), w2),
            pl.BlockSpec((NB, NQ, CC), lambda b, i: (0, 0, 0)),
            pl.BlockSpec((64, 64), w2),
            pl.BlockSpec((192, 192), w2),
            pl.BlockSpec((320, 320), w2),
        ],
        out_specs=[
            pl.BlockSpec((1, N, CMT), full),
            pl.BlockSpec((1, 3, N), full),
        ],
        out_shape=[
            jax.ShapeDtypeStruct((B, N, CMT), jnp.float32),
            jax.ShapeDtypeStruct((B, 3, N), jnp.float32),
        ],
    )(posn, post, nmaskf, F, FT, U, A, GN, wexp, vexp, lwt0, lwt1, lwt2)


def _embed(z, embp):
    B, N = z.shape
    return pl.pallas_call(
        _emb_body,
        grid=(B,),
        in_specs=[
            pl.BlockSpec((1, N, 1), lambda b: (b, 0, 0)),
            pl.BlockSpec((128, CC), lambda b: (0, 0)),
        ],
        out_specs=pl.BlockSpec((1, N, CMT), lambda b: (b, 0, 0)),
        out_shape=jax.ShapeDtypeStruct((B, N, CMT), jnp.float32),
    )(z.reshape(B, N, 1).astype(jnp.int32), embp)


def _prep_weights(params):
    wexp, vexp, lbs = [], [], []
    blocks = []
    for blk in params["blocks"]:
        we = jnp.concatenate([jnp.tile(blk["W"][l], (1, MS[l])) for l in range(3)], axis=1)
        ve = jnp.stack([blk["V"][0]] + [blk["V"][1]] * 3 + [blk["V"][2]] * 5, axis=1)
        lw, lwt, lb = [], [], []
        for l in range(3):
            M = MS[l]
            perm = (jnp.arange(CC)[None, :] * M + jnp.arange(M)[:, None]).reshape(-1)
            w = blk["lw"][l][perm][:, perm]
            lw.append(w)
            lwt.append(w.T)
            lb.append(blk["lb"][l][perm])
        blocks.append(dict(wexp=we, vexp=ve, lw=lw, lwt=lwt,
                           lb=jnp.concatenate(lb).reshape(1, CMT)))
    return blocks


def kernel(z, pos, neighbor_mask, params):
    B, N, _ = pos.shape
    posn = pos.astype(jnp.float32)
    post = jnp.transpose(posn, (0, 2, 1))
    nmaskf = neighbor_mask.astype(jnp.float32)
    blocks = _prep_weights(params)
    embp = jnp.concatenate(
        [params["emb"], jnp.zeros((128 - params["emb"].shape[0], CC), jnp.float32)], axis=0)
    ow = params["out_w"].reshape(1, CC)

    F = _embed(z, embp)
    saves = []
    e = None
    for t in range(3):
        bw = blocks[t]
        Fn, U, A, e = _fwd_layer(posn, post, nmaskf, F, bw["wexp"], bw["vexp"],
                                 bw["lw"][0], bw["lw"][1], bw["lw"][2], bw["lb"], ow)
        saves.append((F, U, A))
        F = Fn
    E = e[:, 0, 0] + N * params["out_b"][0]

    GN = jnp.concatenate(
        [jnp.broadcast_to(params["out_w"][:, 0][None, None, :], (B, N, CC)),
         jnp.zeros((B, N, CMT - CC), jnp.float32)], axis=2)
    gpt_sum = jnp.zeros((B, 3, N), jnp.float32)
    for t in (2, 1, 0):
        bw = blocks[t]
        Fin, U, A = saves[t]
        FT = jnp.transpose(Fin, (0, 2, 1))
        GN, gpt = _bwd_layer(posn, post, nmaskf, Fin, FT, U, A, GN,
                             bw["wexp"], bw["vexp"],
                             bw["lwt"][0], bw["lwt"][1], bw["lwt"][2])
        gpt_sum = gpt_sum + gpt
    Fforce = -jnp.transpose(gpt_sum, (0, 2, 1))
    return (E, Fforce)


# speedup vs baseline: 4.0392x; 1.2293x over previous
import jax
import jax.numpy as jnp
import numpy as np
from jax import lax
from jax.experimental import pallas as pl

RC = 5.0
NB = 16
CC = 64
NQ = 9
CMT = 576
BI = 32
OFFS = (0, 64, 256, 576)
MS = (1, 3, 5)
WID = RC / NB


def _silu(x):
    return x * jax.nn.sigmoid(x)


def _dsilu(x):
    s = jax.nn.sigmoid(x)
    return s * (1.0 + x * (1.0 - s))


def _pair_quantities(posn_ref, post, nmask, i0, bi, n):
    xi = posn_ref[0, pl.ds(i0, bi), 0:1]
    yi = posn_ref[0, pl.ds(i0, bi), 1:2]
    zi = posn_ref[0, pl.ds(i0, bi), 2:3]
    dx = post[0:1, :] - xi
    dy = post[1:2, :] - yi
    dz = post[2:3, :] - zi
    d2 = dx * dx + dy * dy + dz * dz + 1e-12
    dd = jnp.sqrt(d2)
    inv = 1.0 / dd
    row = lax.broadcasted_iota(jnp.int32, (bi, n), 0) + i0
    col = lax.broadcasted_iota(jnp.int32, (bi, n), 1)
    offdiag = jnp.where(row == col, 0.0, 1.0).astype(jnp.float32)
    incut = jnp.where(dd < RC, 1.0, 0.0).astype(jnp.float32)
    maskf = nmask * offdiag * incut
    env = 0.5 * (jnp.cos(jnp.pi * jnp.minimum(dd / RC, 1.0)) + 1.0) * maskf
    cent = (lax.broadcasted_iota(jnp.int32, (NB, 1, 1), 0).astype(jnp.float32)
            * np.float32(RC / (NB - 1)))
    dev = dd[None, :, :] - cent
    rb = jnp.exp(-(dev * dev) / (2.0 * WID * WID))
    P = rb * env[None, :, :]
    xh = dx * inv
    yh = dy * inv
    zh = dz * inv
    Ys = jnp.concatenate(
        [
            jnp.ones((1, bi, n), jnp.float32),
            xh[None], yh[None], zh[None],
            (xh * yh)[None], (yh * zh)[None],
            (3.0 * zh * zh - 1.0)[None],
            (xh * zh)[None], (xh * xh - yh * yh)[None],
        ],
        axis=0,
    )
    return dict(dd=dd, inv=inv, maskf=maskf, env=env, rb=rb, P=P,
                xh=xh, yh=yh, zh=zh, Ys=Ys, dev=dev)


def _fwd_body(posn_ref, post_ref, nmask_ref, f_ref, wexp_ref, vexp_ref,
              lw0_ref, lw1_ref, lw2_ref, lb_ref, ow_ref,
              fout_ref, u_ref, a_ref, e_ref):
    ib = pl.program_id(1)
    i0 = ib * BI
    post = post_ref[0]
    nm = nmask_ref[0]
    F = f_ref[0]
    n = F.shape[0]
    pq = _pair_quantities(posn_ref, post, nm, i0, BI, n)
    P, Ys = pq["P"], pq["Ys"]
    PY = P[:, None, :, :] * Ys[None, :, :, :]
    G = jnp.dot(P.reshape(NB * BI, n), F,
                preferred_element_type=jnp.float32)
    H = jnp.dot(PY.reshape(NB * NQ * BI, n), F[:, 0:CC],
                preferred_element_type=jnp.float32)
    t1 = jnp.sum(G.reshape(NB, BI, CMT) * wexp_ref[...][:, None, :], axis=0)
    H4 = H.reshape(NB, NQ, BI, CC)
    t2q = jnp.sum(H4 * vexp_ref[...][:, :, None, :], axis=0)
    t2 = jnp.concatenate([t2q[q] for q in range(NQ)], axis=1)
    conv = t1 + t2
    Fi = f_ref[0, pl.ds(i0, BI), :]
    lws = (lw0_ref, lw1_ref, lw2_ref)
    nf0 = None
    for l in range(3):
        sl = slice(OFFS[l], OFFS[l + 1])
        u = jnp.dot(conv[:, sl], lws[l][...],
                    preferred_element_type=jnp.float32) + lb_ref[0:1, sl]
        y = _silu(u)
        a = Fi[:, sl] + y
        nf = _silu(a)
        u_ref[0, :, sl] = u
        a_ref[0, :, sl] = a
        fout_ref[0, :, sl] = nf
        if l == 0:
            nf0 = nf
    eval_ = jnp.sum(nf0 * ow_ref[...])

    @pl.when(ib == 0)
    def _():
        e_ref[...] = jnp.full((1, 1, 1), 0.0, jnp.float32) + eval_

    @pl.when(ib > 0)
    def _():
        e_ref[...] = e_ref[...] + eval_


def _bwd_body(posn_ref, post_ref, nmask_ref, f_ref, ft_ref, u_ref, a_ref,
              gn_ref, wexp_ref, vexp_ref, lwt0_ref, lwt1_ref, lwt2_ref,
              gf_ref, gpt_ref):
    ib = pl.program_id(1)
    i0 = ib * BI
    post = post_ref[0]
    nm = nmask_ref[0]
    F = f_ref[0]
    FT = ft_ref[0]
    n = F.shape[0]
    pq = _pair_quantities(posn_ref, post, nm, i0, BI, n)
    P, Ys = pq["P"], pq["Ys"]

    a = a_ref[0]
    u = u_ref[0]
    gn = gn_ref[0]
    ga = gn * _dsilu(a)
    gu = ga * _dsilu(u)
    lwts = (lwt0_ref, lwt1_ref, lwt2_ref)
    gconv = jnp.concatenate(
        [jnp.dot(gu[:, OFFS[l]:OFFS[l + 1]], lwts[l][...],
                 preferred_element_type=jnp.float32) for l in range(3)],
        axis=1,
    )

    U1 = wexp_ref[...][:, None, :] * gconv[None]
    gP1 = jnp.dot(U1.reshape(NB * BI, CMT), FT,
                  preferred_element_type=jnp.float32)
    gf1 = lax.dot_general(P.reshape(NB * BI, n), U1.reshape(NB * BI, CMT),
                          (((0,), (0,)), ((), ())),
                          preferred_element_type=jnp.float32)
    gconvq = jnp.concatenate(
        [gconv[None, :, q * CC:(q + 1) * CC] for q in range(NQ)], axis=0)
    U2 = vexp_ref[...][:, :, None, :] * gconvq[None]
    Z = jnp.dot(U2.reshape(NB * NQ * BI, CC), FT[0:CC, :],
                preferred_element_type=jnp.float32)
    Z4 = Z.reshape(NB, NQ, BI, n)
    gP = gP1.reshape(NB, BI, n) + jnp.sum(Z4 * Ys[None], axis=1)
    gY = jnp.sum(Z4 * P[:, None, :, :], axis=0)
    PY = P[:, None, :, :] * Ys[None]
    gs = lax.dot_general(PY.reshape(NB * NQ * BI, n), U2.reshape(NB * NQ * BI, CC),
                         (((0,), (0,)), ((), ())),
                         preferred_element_type=jnp.float32)

    dd, inv, env, rb, maskf = pq["dd"], pq["inv"], pq["env"], pq["rb"], pq["maskf"]
    xh, yh, zh, dev = pq["xh"], pq["yh"], pq["zh"], pq["dev"]
    rbp = rb * (-dev / (WID * WID))
    envp = (-0.5 * jnp.pi / RC) * jnp.sin(
        jnp.pi * jnp.minimum(dd / RC, 1.0)) * maskf
    gd = jnp.sum(gP * (rbp * env[None] + rb * envp[None]), axis=0)
    gxh = gY[1] + gY[4] * yh + gY[7] * zh + 2.0 * gY[8] * xh
    gyh = gY[2] + gY[4] * xh + gY[5] * zh - 2.0 * gY[8] * yh
    gzh = gY[3] + gY[5] * yh + 6.0 * gY[6] * zh + gY[7] * xh
    dotg = gxh * xh + gyh * yh + gzh * zh
    grx = gd * xh + (gxh - dotg * xh) * inv
    gry = gd * yh + (gyh - dotg * yh) * inv
    grz = gd * zh + (gzh - dotg * zh) * inv

    ones_bi = jnp.ones((1, BI), jnp.float32)
    ones_n = jnp.ones((1, n), jnp.float32)
    cn = (((1,), (1,)), ((), ()))
    colx = jnp.dot(ones_bi, grx, preferred_element_type=jnp.float32)
    coly = jnp.dot(ones_bi, gry, preferred_element_type=jnp.float32)
    colz = jnp.dot(ones_bi, grz, preferred_element_type=jnp.float32)
    rowx = lax.dot_general(ones_n, grx, cn,
                           preferred_element_type=jnp.float32)
    rowy = lax.dot_general(ones_n, gry, cn, preferred_element_type=jnp.float32)
    rowz = lax.dot_general(ones_n, grz, cn, preferred_element_type=jnp.float32)
    ri = lax.broadcasted_iota(jnp.int32, (BI, n), 0) + i0
    ci = lax.broadcasted_iota(jnp.int32, (BI, n), 1)
    oh = jnp.where(ri == ci, 1.0, 0.0).astype(jnp.float32)
    sx = jnp.dot(rowx, oh, preferred_element_type=jnp.float32)
    sy = jnp.dot(rowy, oh, preferred_element_type=jnp.float32)
    sz = jnp.dot(rowz, oh, preferred_element_type=jnp.float32)

    @pl.when(ib == 0)
    def _():
        gf_ref[0, :, :] = jnp.zeros((n, CMT), jnp.float32)
        gpt_ref[0, :, :] = jnp.zeros((3, n), jnp.float32)

    gf_ref[0, :, :] += gf1
    gf_ref[0, :, 0:CC] += gs
    gf_ref[0, pl.ds(i0, BI), :] += ga
    gpt_ref[0, 0:1, :] += colx - sx
    gpt_ref[0, 1:2, :] += coly - sy
    gpt_ref[0, 2:3, :] += colz - sz


def _emb_body(z_ref, emb_ref, f0_ref):
    zc = z_ref[0]
    n = zc.shape[0]
    io = lax.broadcasted_iota(jnp.int32, (n, 128), 1)
    oh = jnp.where(io == zc, 1.0, 0.0).astype(jnp.float32)
    f0 = jnp.dot(oh, emb_ref[...], preferred_element_type=jnp.float32)
    f0_ref[0, :, :] = jnp.concatenate(
        [f0, jnp.zeros((n, CMT - CC), jnp.float32)], axis=1)


def _fwd_layer(posn, post, nmaskf, F, wexp, vexp, lw0, lw1, lw2, lb, ow):
    B, N = F.shape[0], F.shape[1]
    NI = N // BI
    full = lambda b, i: (b, 0, 0)
    w2 = lambda b, i: (0, 0)
    return pl.pallas_call(
        _fwd_body,
        grid=(B, NI),
        in_specs=[
            pl.BlockSpec((1, N, 3), full),
            pl.BlockSpec((1, 3, N), full),
            pl.BlockSpec((1, BI, N), lambda b, i: (b, i, 0)),
            pl.BlockSpec((1, N, CMT), full),
            pl.BlockSpec((NB, CMT), w2),
            pl.BlockSpec((NB, NQ, CC), lambda b, i: (0, 0, 0)),
            pl.BlockSpec((64, 64), w2),
            pl.BlockSpec((192, 192), w2),
            pl.BlockSpec((320, 320), w2),
            pl.BlockSpec((1, CMT), w2),
            pl.BlockSpec((1, CC), w2),
        ],
        out_specs=[
            pl.BlockSpec((1, BI, CMT), lambda b, i: (b, i, 0)),
            pl.BlockSpec((1, BI, CMT), lambda b, i: (b, i, 0)),
            pl.BlockSpec((1, BI, CMT), lambda b, i: (b, i, 0)),
            pl.BlockSpec((1, 1, 1), full),
        ],
        out_shape=[
            jax.ShapeDtypeStruct((B, N, CMT), jnp.float32),
            jax.ShapeDtypeStruct((B, N, CMT), jnp.float32),
            jax.ShapeDtypeStruct((B, N, CMT), jnp.float32),
            jax.ShapeDtypeStruct((B, 1, 1), jnp.float32),
        ],
    )(posn, post, nmaskf, F, wexp, vexp, lw0, lw1, lw2, lb, ow)


def _bwd_layer(posn, post, nmaskf, F, FT, U, A, GN, wexp, vexp, lwt0, lwt1, lwt2):
    B, N = F.shape[0], F.shape[1]
    NI = N // BI
    full = lambda b, i: (b, 0, 0)
    blk = lambda b, i: (b, i, 0)
    w2 = lambda b, i: (0, 0)
    return pl.pallas_call(
        _bwd_body,
        grid=(B, NI),
        in_specs=[
            pl.BlockSpec((1, N, 3), full),
            pl.BlockSpec((1, 3, N), full),
            pl.BlockSpec((1, BI, N), blk),
            pl.BlockSpec((1, N, CMT), full),
            pl.BlockSpec((1, CMT, N), full),
            pl.BlockSpec((1, BI, CMT), blk),
            pl.BlockSpec((1, BI, CMT), blk),
            pl.BlockSpec((1, BI, CMT), blk),
            pl.BlockSpec((NB, CMT), w2),
            pl.BlockSpec((NB, NQ, CC), lambda b, i: (0, 0, 0)),
            pl.BlockSpec((64, 64), w2),
            pl.BlockSpec((192, 192), w2),
            pl.BlockSpec((320, 320), w2),
        ],
        out_specs=[
            pl.BlockSpec((1, N, CMT), full),
            pl.BlockSpec((1, 3, N), full),
        ],
        out_shape=[
            jax.ShapeDtypeStruct((B, N, CMT), jnp.float32),
            jax.ShapeDtypeStruct((B, 3, N), jnp.float32),
        ],
    )(posn, post, nmaskf, F, FT, U, A, GN, wexp, vexp, lwt0, lwt1, lwt2)


def _embed(z, embp):
    B, N = z.shape
    return pl.pallas_call(
        _emb_body,
        grid=(B,),
        in_specs=[
            pl.BlockSpec((1, N, 1), lambda b: (b, 0, 0)),
            pl.BlockSpec((128, CC), lambda b: (0, 0)),
        ],
        out_specs=pl.BlockSpec((1, N, CMT), lambda b: (b, 0, 0)),
        out_shape=jax.ShapeDtypeStruct((B, N, CMT), jnp.float32),
    )(z.reshape(B, N, 1).astype(jnp.int32), embp)


def _prep_weights(params):
    wexp, vexp, lbs = [], [], []
    blocks = []
    for blk in params["blocks"]:
        we = jnp.concatenate([jnp.tile(blk["W"][l], (1, MS[l])) for l in range(3)], axis=1)
        ve = jnp.stack([blk["V"][0]] + [blk["V"][1]] * 3 + [blk["V"][2]] * 5, axis=1)
        lw, lwt, lb = [], [], []
        for l in range(3):
            M = MS[l]
            perm = (jnp.arange(CC)[None, :] * M + jnp.arange(M)[:, None]).reshape(-1)
            w = blk["lw"][l][perm][:, perm]
            lw.append(w)
            lwt.append(w.T)
            lb.append(blk["lb"][l][perm])
        blocks.append(dict(wexp=we, vexp=ve, lw=lw, lwt=lwt,
                           lb=jnp.concatenate(lb).reshape(1, CMT)))
    return blocks


def kernel(z, pos, neighbor_mask, params):
    B, N, _ = pos.shape
    posn = pos.astype(jnp.float32)
    post = jnp.transpose(posn, (0, 2, 1))
    nmaskf = neighbor_mask.astype(jnp.float32)
    blocks = _prep_weights(params)
    embp = jnp.concatenate(
        [params["emb"], jnp.zeros((128 - params["emb"].shape[0], CC), jnp.float32)], axis=0)
    ow = params["out_w"].reshape(1, CC)

    F = _embed(z, embp)
    saves = []
    e = None
    for t in range(3):
        bw = blocks[t]
        Fn, U, A, e = _fwd_layer(posn, post, nmaskf, F, bw["wexp"], bw["vexp"],
                                 bw["lw"][0], bw["lw"][1], bw["lw"][2], bw["lb"], ow)
        saves.append((F, U, A))
        F = Fn
    E = e[:, 0, 0] + N * params["out_b"][0]

    GN = jnp.concatenate(
        [jnp.broadcast_to(params["out_w"][:, 0][None, None, :], (B, N, CC)),
         jnp.zeros((B, N, CMT - CC), jnp.float32)], axis=2)
    gpt_sum = jnp.zeros((B, 3, N), jnp.float32)
    for t in (2, 1, 0):
        bw = blocks[t]
        Fin, U, A = saves[t]
        FT = jnp.transpose(Fin, (0, 2, 1))
        GN, gpt = _bwd_layer(posn, post, nmaskf, Fin, FT, U, A, GN,
                             bw["wexp"], bw["vexp"],
                             bw["lwt"][0], bw["lwt"][1], bw["lwt"][2])
        gpt_sum = gpt_sum + gpt
    Fforce = -jnp.transpose(gpt_sum, (0, 2, 1))
    return (E, Fforce)
